# baseline (speedup 1.0000x reference)
import functools
import math

import jax
import jax.numpy as jnp
from jax import lax
from jax.experimental import pallas as pl
from jax.experimental.pallas import tpu as pltpu
from jax.experimental.pallas import tpu_sc as plsc

_TOKEN_BLOCK = 256
_GATE_PAD = 128
_SC_WORKERS = 32
_SC_CHUNK = 64


def _router_body(n_tok, n_exp, cap, aux_coef,
                 x_ref, wr_ref, xa_ref, scat_ref, gath_ref, aux_ref,
                 counts_ref, imp_ref):
    i = pl.program_id(0)
    tb = x_ref.shape[0]

    @pl.when(i == 0)
    def _():
        counts_ref[...] = jnp.zeros_like(counts_ref)
        imp_ref[...] = jnp.zeros_like(imp_ref)
        aux_ref[...] = jnp.zeros_like(aux_ref)

    x = x_ref[...]
    logits = jnp.dot(x, wr_ref[...], preferred_element_type=jnp.float32)
    m = jnp.max(logits, axis=-1, keepdims=True)
    e = jnp.exp(logits - m)
    probs = e / jnp.sum(e, axis=-1, keepdims=True)
    gate = jnp.max(probs, axis=-1, keepdims=True)
    lane = lax.broadcasted_iota(jnp.int32, (tb, n_exp), 1)
    eidx = jnp.min(jnp.where(logits == m, lane, n_exp), axis=-1,
                   keepdims=True)
    onehot = (lane == eidx).astype(jnp.float32)

    row = lax.broadcasted_iota(jnp.int32, (tb, tb), 0)
    col = lax.broadcasted_iota(jnp.int32, (tb, tb), 1)
    tril = (col < row).astype(jnp.float32)
    prefix = jnp.dot(tril, onehot, preferred_element_type=jnp.float32)
    counts_prev = counts_ref[...]
    rank = jnp.sum(onehot * (prefix + counts_prev), axis=-1,
                   keepdims=True).astype(jnp.int32)
    counts_ref[...] = counts_prev + jnp.sum(onehot, axis=0, keepdims=True)
    imp_ref[...] = imp_ref[...] + jnp.sum(probs, axis=0, keepdims=True)

    slot = eidx * cap + rank
    valid = rank < cap
    trash = n_exp * cap
    scat_ref[...] = jnp.where(valid, slot, trash)
    gath_ref[...] = jnp.where(valid, slot, trash)

    d = x.shape[1]
    xa_ref[:, :d] = x
    pad = jnp.zeros((tb, _GATE_PAD - 1), jnp.float32)
    xa_ref[:, d:] = jnp.concatenate([gate, pad], axis=1)

    @pl.when(i == pl.num_programs(0) - 1)
    def _():
        lbl = jnp.sum(imp_ref[...] * counts_ref[...], axis=(0, 1),
                      keepdims=True)
        aux_ref[...] = aux_coef * lbl * (n_exp / (float(n_tok) * n_tok))


def _router(x_flat, wr, cap, aux_coef, interpret=False):
    n_tok, d = x_flat.shape
    n_exp = wr.shape[1]
    tb = _TOKEN_BLOCK
    da = d + _GATE_PAD
    grid = n_tok // tb
    body = functools.partial(_router_body, n_tok, n_exp, cap, aux_coef)
    return pl.pallas_call(
        body,
        grid=(grid,),
        in_specs=[
            pl.BlockSpec((tb, d), lambda i: (i, 0)),
            pl.BlockSpec((d, n_exp), lambda i: (0, 0)),
        ],
        out_specs=[
            pl.BlockSpec((tb, da), lambda i: (i, 0)),
            pl.BlockSpec((tb, 1), lambda i: (i, 0)),
            pl.BlockSpec((tb, 1), lambda i: (i, 0)),
            pl.BlockSpec((1, 1), lambda i: (0, 0)),
        ],
        out_shape=[
            jax.ShapeDtypeStruct((n_tok, da), jnp.float32),
            jax.ShapeDtypeStruct((n_tok, 1), jnp.int32),
            jax.ShapeDtypeStruct((n_tok, 1), jnp.int32),
            jax.ShapeDtypeStruct((1, 1), jnp.float32),
        ],
        scratch_shapes=[
            pltpu.VMEM((1, n_exp), jnp.float32),
            pltpu.VMEM((1, n_exp), jnp.float32),
        ],
        interpret=interpret,
    )(x_flat, wr)


def _dispatch(xa, scat_idx3, n_slot):
    n_tok, da = xa.shape
    per_w = n_tok // _SC_WORKERS
    chunk = min(_SC_CHUNK, per_w)
    n_chunk = per_w // chunk
    mesh = plsc.VectorSubcoreMesh(core_axis_name="c", subcore_axis_name="s")

    @functools.partial(
        pl.kernel,
        out_type=jax.ShapeDtypeStruct((n_slot, da), jnp.float32),
        mesh=mesh,
        scratch_types=[
            pltpu.VMEM((n_chunk, chunk), jnp.int32),
            pltpu.VMEM((chunk, da), jnp.float32),
            pltpu.VMEM((chunk, da), jnp.float32),
            pltpu.SemaphoreType.DMA,
            pltpu.SemaphoreType.DMA,
            pltpu.SemaphoreType.DMA,
            pltpu.SemaphoreType.DMA,
        ],
    )
    def k(x_hbm, idx_hbm, xg_hbm, idx_v, buf_a, buf_b, sia, sib, soa, sob):
        wid = lax.axis_index("s") * 2 + lax.axis_index("c")
        base = wid * per_w
        pltpu.sync_copy(idx_hbm.at[wid], idx_v)

        bufs = (buf_a, buf_b)
        sin = (sia, sib)
        sout = (soa, sob)

        def cp_in(c):
            return pltpu.async_copy(
                x_hbm.at[pl.ds(base + c * chunk, chunk)],
                bufs[c % 2], sin[c % 2])

        def cp_out(c):
            return pltpu.async_copy(
                bufs[c % 2], xg_hbm.at[idx_v.at[c]], sout[c % 2])

        pend_in = {0: cp_in(0)}
        pend_out = {}
        for c in range(n_chunk):
            pend_in.pop(c).wait()
            if c + 1 < n_chunk:
                if c >= 1:
                    pend_out.pop(c - 1).wait()
                pend_in[c + 1] = cp_in(c + 1)
            pend_out[c] = cp_out(c)
        for c in sorted(pend_out):
            pend_out.pop(c).wait()

    return k(xa, scat_idx3)


def _collect(yg, gath_idx3, n_tok):
    d = yg.shape[1]
    per_w = n_tok // _SC_WORKERS
    chunk = min(_SC_CHUNK, per_w)
    n_chunk = per_w // chunk
    mesh = plsc.VectorSubcoreMesh(core_axis_name="c", subcore_axis_name="s")

    @functools.partial(
        pl.kernel,
        out_type=jax.ShapeDtypeStruct((n_tok, d), jnp.float32),
        mesh=mesh,
        scratch_types=[
            pltpu.VMEM((n_chunk, chunk), jnp.int32),
            pltpu.VMEM((chunk, d), jnp.float32),
            pltpu.VMEM((chunk, d), jnp.float32),
            pltpu.SemaphoreType.DMA,
            pltpu.SemaphoreType.DMA,
            pltpu.SemaphoreType.DMA,
            pltpu.SemaphoreType.DMA,
        ],
    )
    def k(yg_hbm, idx_hbm, ys_hbm, idx_v, buf_a, buf_b, sia, sib, soa, sob):
        wid = lax.axis_index("s") * 2 + lax.axis_index("c")
        base = wid * per_w
        pltpu.sync_copy(idx_hbm.at[wid], idx_v)

        bufs = (buf_a, buf_b)
        sin = (sia, sib)
        sout = (soa, sob)

        def cp_in(c):
            return pltpu.async_copy(
                yg_hbm.at[idx_v.at[c]], bufs[c % 2], sin[c % 2])

        def cp_out(c):
            return pltpu.async_copy(
                bufs[c % 2], ys_hbm.at[pl.ds(base + c * chunk, chunk)],
                sout[c % 2])

        pend_in = {0: cp_in(0)}
        pend_out = {}
        for c in range(n_chunk):
            pend_in.pop(c).wait()
            if c + 1 < n_chunk:
                if c >= 1:
                    pend_out.pop(c - 1).wait()
                pend_in[c + 1] = cp_in(c + 1)
            pend_out[c] = cp_out(c)
        for c in sorted(pend_out):
            pend_out.pop(c).wait()

    return k(yg, gath_idx3)


def _ffn_body(n_exp, d, cap, epb, xg_ref, wg_ref, wu_ref, wd_ref, yg_ref):
    e = pl.program_id(0)

    @pl.when(e < n_exp // epb)
    def _():
        for k in range(epb):
            rows = pl.ds(k * cap, cap)
            xe = xg_ref[rows, :d].astype(jnp.bfloat16)
            gate = xg_ref[rows, d:d + 1]
            a = jnp.dot(xe, wg_ref[k].astype(jnp.bfloat16),
                        preferred_element_type=jnp.float32)
            b = jnp.dot(xe, wu_ref[k].astype(jnp.bfloat16),
                        preferred_element_type=jnp.float32)
            h = (a * jax.nn.sigmoid(a)) * b
            y = jnp.dot(h.astype(jnp.bfloat16),
                        wd_ref[k].astype(jnp.bfloat16),
                        preferred_element_type=jnp.float32)
            yg_ref[rows, :] = y * gate

    @pl.when(e == n_exp // epb)
    def _():
        yg_ref[...] = jnp.zeros_like(yg_ref)


def _ffn(xg, wg, wu, wd, cap, interpret=False):
    n_exp, d, dff = wg.shape
    da = xg.shape[1]
    epb = 4
    body = functools.partial(_ffn_body, n_exp, d, cap, epb)
    last = n_exp // epb - 1
    return pl.pallas_call(
        body,
        grid=(n_exp // epb + 1,),
        in_specs=[
            pl.BlockSpec((epb * cap, da), lambda e: (e, 0)),
            pl.BlockSpec((epb, d, dff), lambda e: (jnp.minimum(e, last), 0, 0)),
            pl.BlockSpec((epb, d, dff), lambda e: (jnp.minimum(e, last), 0, 0)),
            pl.BlockSpec((epb, dff, d), lambda e: (jnp.minimum(e, last), 0, 0)),
        ],
        out_specs=pl.BlockSpec((epb * cap, d), lambda e: (e, 0)),
        out_shape=jax.ShapeDtypeStruct(((n_exp + epb) * cap, d), jnp.float32),
        interpret=interpret,
    )(xg, wg, wu, wd)


def kernel(x, Wr, Wg, Wu, Wd):
    b, s, d = x.shape
    n_tok = b * s
    n_exp = Wr.shape[1]
    cap = max(1, int(math.ceil(1.25 * (n_tok / n_exp))))
    x_flat = x.reshape(n_tok, d)

    xa, scat2, gath2, aux = _router(x_flat, Wr, cap, 0.01)
    per_w = n_tok // _SC_WORKERS
    chunk = min(_SC_CHUNK, per_w)
    scat3 = scat2.reshape(_SC_WORKERS, per_w // chunk, chunk)
    gath3 = gath2.reshape(_SC_WORKERS, per_w // chunk, chunk)

    n_slot = (n_exp + 4) * cap
    xg = _dispatch(xa, scat3, n_slot)
    yg = _ffn(xg, Wg, Wu, Wd, cap)
    out = _collect(yg, gath3, n_tok)
    return out.reshape(b, s, d), aux.reshape(())


# speedup vs baseline: 1.0782x; 1.0782x over previous
import functools
import math

import jax
import jax.numpy as jnp
from jax import lax
from jax.experimental import pallas as pl
from jax.experimental.pallas import tpu as pltpu
from jax.experimental.pallas import tpu_sc as plsc

_TOKEN_BLOCK = 512
_GATE_W = 128
_SC_WORKERS = 32
_SC_CHUNK = 64


def _router_body(n_tok, n_exp, cap, aux_coef,
                 x_ref, wr_ref, g_ref, scat_ref, gath_ref, aux_ref,
                 counts_ref, imp_ref):
    i = pl.program_id(0)
    tb = x_ref.shape[0]

    @pl.when(i == 0)
    def _():
        counts_ref[...] = jnp.zeros_like(counts_ref)
        imp_ref[...] = jnp.zeros_like(imp_ref)
        aux_ref[...] = jnp.zeros_like(aux_ref)

    x = x_ref[...]
    logits = jnp.dot(x, wr_ref[...], preferred_element_type=jnp.float32)
    m = jnp.max(logits, axis=-1, keepdims=True)
    e = jnp.exp(logits - m)
    probs = e / jnp.sum(e, axis=-1, keepdims=True)
    gate = jnp.max(probs, axis=-1, keepdims=True)
    lane = lax.broadcasted_iota(jnp.int32, (tb, n_exp), 1)
    eidx = jnp.min(jnp.where(logits == m, lane, n_exp), axis=-1,
                   keepdims=True)
    onehot = (lane == eidx).astype(jnp.float32)

    row = lax.broadcasted_iota(jnp.int32, (tb, tb), 0)
    col = lax.broadcasted_iota(jnp.int32, (tb, tb), 1)
    tril = (col < row).astype(jnp.float32)
    prefix = jnp.dot(tril, onehot, preferred_element_type=jnp.float32)
    counts_prev = counts_ref[...]
    rank = jnp.sum(onehot * (prefix + counts_prev), axis=-1,
                   keepdims=True).astype(jnp.int32)
    counts_ref[...] = counts_prev + jnp.sum(onehot, axis=0, keepdims=True)
    imp_ref[...] = imp_ref[...] + jnp.sum(probs, axis=0, keepdims=True)

    slot = eidx * cap + rank
    valid = rank < cap
    trash = n_exp * cap
    scat_ref[...] = jnp.where(valid, slot, trash)
    gath_ref[...] = jnp.where(valid, slot, trash)

    pad = jnp.zeros((tb, _GATE_W - 1), jnp.float32)
    g_ref[...] = jnp.concatenate([gate, pad], axis=1)

    @pl.when(i == pl.num_programs(0) - 1)
    def _():
        lbl = jnp.sum(imp_ref[...] * counts_ref[...], axis=(0, 1),
                      keepdims=True)
        aux_ref[...] = aux_coef * lbl * (n_exp / (float(n_tok) * n_tok))


def _router(x_flat, wr, cap, aux_coef, interpret=False):
    n_tok, d = x_flat.shape
    n_exp = wr.shape[1]
    tb = _TOKEN_BLOCK
    grid = n_tok // tb
    body = functools.partial(_router_body, n_tok, n_exp, cap, aux_coef)
    return pl.pallas_call(
        body,
        grid=(grid,),
        in_specs=[
            pl.BlockSpec((tb, d), lambda i: (i, 0)),
            pl.BlockSpec((d, n_exp), lambda i: (0, 0)),
        ],
        out_specs=[
            pl.BlockSpec((tb, _GATE_W), lambda i: (i, 0)),
            pl.BlockSpec((tb, 1), lambda i: (i, 0)),
            pl.BlockSpec((tb, 1), lambda i: (i, 0)),
            pl.BlockSpec((1, 1), lambda i: (0, 0)),
        ],
        out_shape=[
            jax.ShapeDtypeStruct((n_tok, _GATE_W), jnp.float32),
            jax.ShapeDtypeStruct((n_tok, 1), jnp.int32),
            jax.ShapeDtypeStruct((n_tok, 1), jnp.int32),
            jax.ShapeDtypeStruct((1, 1), jnp.float32),
        ],
        scratch_shapes=[
            pltpu.VMEM((1, n_exp), jnp.float32),
            pltpu.VMEM((1, n_exp), jnp.float32),
        ],
        interpret=interpret,
    )(x_flat, wr)


def _dispatch(x_flat, gates, scat_idx3, n_slot):
    n_tok, d = x_flat.shape
    per_w = n_tok // _SC_WORKERS
    chunk = min(_SC_CHUNK, per_w)
    n_chunk = per_w // chunk
    mesh = plsc.VectorSubcoreMesh(core_axis_name="c", subcore_axis_name="s")

    @functools.partial(
        pl.kernel,
        out_type=[
            jax.ShapeDtypeStruct((n_slot, d), jnp.float32),
            jax.ShapeDtypeStruct((n_slot, _GATE_W), jnp.float32),
        ],
        mesh=mesh,
        scratch_types=[
            pltpu.VMEM((n_chunk, chunk), jnp.int32),
            pltpu.VMEM((chunk, d), jnp.float32),
            pltpu.VMEM((chunk, d), jnp.float32),
            pltpu.VMEM((chunk, _GATE_W), jnp.float32),
            pltpu.VMEM((chunk, _GATE_W), jnp.float32),
            pltpu.SemaphoreType.DMA,
            pltpu.SemaphoreType.DMA,
            pltpu.SemaphoreType.DMA,
            pltpu.SemaphoreType.DMA,
        ],
    )
    def k(x_hbm, g_hbm, idx_hbm, xg_hbm, gs_hbm, idx_v,
          buf_a, buf_b, gbuf_a, gbuf_b, sia, sib, soa, sob):
        wid = lax.axis_index("s") * 2 + lax.axis_index("c")
        base = wid * per_w
        pltpu.sync_copy(idx_hbm.at[wid], idx_v)

        bufs = (buf_a, buf_b)
        gbufs = (gbuf_a, gbuf_b)
        sin = (sia, sib)
        sout = (soa, sob)

        def cp_in(c):
            sl = pl.ds(base + c * chunk, chunk)
            h1 = pltpu.async_copy(x_hbm.at[sl], bufs[c % 2], sin[c % 2])
            h2 = pltpu.async_copy(g_hbm.at[sl], gbufs[c % 2], sin[c % 2])
            return (h1, h2)

        def cp_out(c):
            h1 = pltpu.async_copy(bufs[c % 2], xg_hbm.at[idx_v.at[c]],
                                  sout[c % 2])
            h2 = pltpu.async_copy(gbufs[c % 2], gs_hbm.at[idx_v.at[c]],
                                  sout[c % 2])
            return (h1, h2)

        def wait(hs):
            for h in hs:
                h.wait()

        pend_in = {0: cp_in(0)}
        pend_out = {}
        for c in range(n_chunk):
            wait(pend_in.pop(c))
            if c + 1 < n_chunk:
                if c >= 1:
                    wait(pend_out.pop(c - 1))
                pend_in[c + 1] = cp_in(c + 1)
            pend_out[c] = cp_out(c)
        for c in sorted(pend_out):
            wait(pend_out.pop(c))

    return k(x_flat, gates, scat_idx3)


def _collect(yg, gath_idx3, n_tok):
    d = yg.shape[1]
    per_w = n_tok // _SC_WORKERS
    chunk = min(_SC_CHUNK, per_w)
    n_chunk = per_w // chunk
    mesh = plsc.VectorSubcoreMesh(core_axis_name="c", subcore_axis_name="s")

    @functools.partial(
        pl.kernel,
        out_type=jax.ShapeDtypeStruct((n_tok, d), jnp.float32),
        mesh=mesh,
        scratch_types=[
            pltpu.VMEM((n_chunk, chunk), jnp.int32),
            pltpu.VMEM((chunk, d), jnp.float32),
            pltpu.VMEM((chunk, d), jnp.float32),
            pltpu.SemaphoreType.DMA,
            pltpu.SemaphoreType.DMA,
            pltpu.SemaphoreType.DMA,
            pltpu.SemaphoreType.DMA,
        ],
    )
    def k(yg_hbm, idx_hbm, ys_hbm, idx_v, buf_a, buf_b, sia, sib, soa, sob):
        wid = lax.axis_index("s") * 2 + lax.axis_index("c")
        base = wid * per_w
        pltpu.sync_copy(idx_hbm.at[wid], idx_v)

        bufs = (buf_a, buf_b)
        sin = (sia, sib)
        sout = (soa, sob)

        def cp_in(c):
            return pltpu.async_copy(
                yg_hbm.at[idx_v.at[c]], bufs[c % 2], sin[c % 2])

        def cp_out(c):
            return pltpu.async_copy(
                bufs[c % 2], ys_hbm.at[pl.ds(base + c * chunk, chunk)],
                sout[c % 2])

        pend_in = {0: cp_in(0)}
        pend_out = {}
        for c in range(n_chunk):
            pend_in.pop(c).wait()
            if c + 1 < n_chunk:
                if c >= 1:
                    pend_out.pop(c - 1).wait()
                pend_in[c + 1] = cp_in(c + 1)
            pend_out[c] = cp_out(c)
        for c in sorted(pend_out):
            pend_out.pop(c).wait()

    return k(yg, gath_idx3)


def _ffn_body(n_exp, d, cap, epb, xg_ref, gs_ref, wg_ref, wu_ref, wd_ref,
              yg_ref):
    e = pl.program_id(0)

    @pl.when(e < n_exp // epb)
    def _():
        for k in range(epb):
            rows = pl.ds(k * cap, cap)
            xe = xg_ref[rows, :].astype(jnp.bfloat16)
            gate = gs_ref[rows, 0:1]
            a = jnp.dot(xe, wg_ref[k].astype(jnp.bfloat16),
                        preferred_element_type=jnp.float32)
            b = jnp.dot(xe, wu_ref[k].astype(jnp.bfloat16),
                        preferred_element_type=jnp.float32)
            h = (a * jax.nn.sigmoid(a)) * b
            y = jnp.dot(h.astype(jnp.bfloat16),
                        wd_ref[k].astype(jnp.bfloat16),
                        preferred_element_type=jnp.float32)
            yg_ref[rows, :] = y * gate

    @pl.when(e == n_exp // epb)
    def _():
        yg_ref[...] = jnp.zeros_like(yg_ref)


def _ffn(xg, gs, wg, wu, wd, cap, interpret=False):
    n_exp, d, dff = wg.shape
    epb = 2
    body = functools.partial(_ffn_body, n_exp, d, cap, epb)
    last = n_exp // epb - 1
    return pl.pallas_call(
        body,
        grid=(n_exp // epb + 1,),
        in_specs=[
            pl.BlockSpec((epb * cap, d), lambda e: (e, 0)),
            pl.BlockSpec((epb * cap, _GATE_W), lambda e: (e, 0)),
            pl.BlockSpec((epb, d, dff), lambda e: (jnp.minimum(e, last), 0, 0)),
            pl.BlockSpec((epb, d, dff), lambda e: (jnp.minimum(e, last), 0, 0)),
            pl.BlockSpec((epb, dff, d), lambda e: (jnp.minimum(e, last), 0, 0)),
        ],
        out_specs=pl.BlockSpec((epb * cap, d), lambda e: (e, 0)),
        out_shape=jax.ShapeDtypeStruct(((n_exp + epb) * cap, d), jnp.float32),
        interpret=interpret,
    )(xg, gs, wg, wu, wd)


def kernel(x, Wr, Wg, Wu, Wd):
    b, s, d = x.shape
    n_tok = b * s
    n_exp = Wr.shape[1]
    cap = max(1, int(math.ceil(1.25 * (n_tok / n_exp))))
    x_flat = x.reshape(n_tok, d)

    gates, scat2, gath2, aux = _router(x_flat, Wr, cap, 0.01)
    per_w = n_tok // _SC_WORKERS
    chunk = min(_SC_CHUNK, per_w)
    scat3 = scat2.reshape(_SC_WORKERS, per_w // chunk, chunk)
    gath3 = gath2.reshape(_SC_WORKERS, per_w // chunk, chunk)

    n_slot = (n_exp + 2) * cap
    xg, gs = _dispatch(x_flat, gates, scat3, n_slot)
    yg = _ffn(xg, gs, Wg, Wu, Wd, cap)
    out = _collect(yg, gath3, n_tok)
    return out.reshape(b, s, d), aux.reshape(())


# speedup vs baseline: 1.1113x; 1.0307x over previous
import functools
import math

import jax
import jax.numpy as jnp
from jax import lax
from jax.experimental import pallas as pl
from jax.experimental.pallas import tpu as pltpu
from jax.experimental.pallas import tpu_sc as plsc

_TOKEN_BLOCK = 1024
_GATE_W = 128
_SC_WORKERS = 32
_SC_CHUNK = 64


def _router_body(n_tok, n_exp, cap, aux_coef,
                 x_ref, wr_ref, g_ref, scat_ref, aux_ref,
                 counts_ref, imp_ref):
    i = pl.program_id(0)
    tb = x_ref.shape[0]

    @pl.when(i == 0)
    def _():
        counts_ref[...] = jnp.zeros_like(counts_ref)
        imp_ref[...] = jnp.zeros_like(imp_ref)
        aux_ref[...] = jnp.zeros_like(aux_ref)

    x = x_ref[...]
    logits = jnp.dot(x, wr_ref[...], preferred_element_type=jnp.float32)
    m = jnp.max(logits, axis=-1, keepdims=True)
    e = jnp.exp(logits - m)
    probs = e / jnp.sum(e, axis=-1, keepdims=True)
    gate = jnp.max(probs, axis=-1, keepdims=True)
    lane = lax.broadcasted_iota(jnp.int32, (tb, n_exp), 1)
    eidx = jnp.min(jnp.where(logits == m, lane, n_exp), axis=-1,
                   keepdims=True)
    onehot = (lane == eidx).astype(jnp.float32)

    row = lax.broadcasted_iota(jnp.int32, (tb, tb), 0)
    col = lax.broadcasted_iota(jnp.int32, (tb, tb), 1)
    tril = (col < row).astype(jnp.float32)
    prefix = jnp.dot(tril, onehot, preferred_element_type=jnp.float32)
    counts_prev = counts_ref[...]
    rank = jnp.sum(onehot * (prefix + counts_prev), axis=-1,
                   keepdims=True).astype(jnp.int32)
    counts_ref[...] = counts_prev + jnp.sum(onehot, axis=0, keepdims=True)
    imp_ref[...] = imp_ref[...] + jnp.sum(probs, axis=0, keepdims=True)

    slot = eidx * cap + rank
    valid = rank < cap
    trash = n_exp * cap
    scat_ref[...] = jnp.where(valid, slot, trash)

    pad = jnp.zeros((tb, _GATE_W - 1), jnp.float32)
    g_ref[...] = jnp.concatenate([gate, pad], axis=1)

    @pl.when(i == pl.num_programs(0) - 1)
    def _():
        lbl = jnp.sum(imp_ref[...] * counts_ref[...], axis=(0, 1),
                      keepdims=True)
        aux_ref[...] = aux_coef * lbl * (n_exp / (float(n_tok) * n_tok))


def _router(x_flat, wr, cap, aux_coef, interpret=False):
    n_tok, d = x_flat.shape
    n_exp = wr.shape[1]
    tb = _TOKEN_BLOCK
    grid = n_tok // tb
    body = functools.partial(_router_body, n_tok, n_exp, cap, aux_coef)
    return pl.pallas_call(
        body,
        grid=(grid,),
        in_specs=[
            pl.BlockSpec((tb, d), lambda i: (i, 0)),
            pl.BlockSpec((d, n_exp), lambda i: (0, 0)),
        ],
        out_specs=[
            pl.BlockSpec((tb, _GATE_W), lambda i: (i, 0)),
            pl.BlockSpec((tb, 1), lambda i: (i, 0)),
            pl.BlockSpec((1, 1), lambda i: (0, 0)),
        ],
        out_shape=[
            jax.ShapeDtypeStruct((n_tok, _GATE_W), jnp.float32),
            jax.ShapeDtypeStruct((n_tok, 1), jnp.int32),
            jax.ShapeDtypeStruct((1, 1), jnp.float32),
        ],
        scratch_shapes=[
            pltpu.VMEM((1, n_exp), jnp.float32),
            pltpu.VMEM((1, n_exp), jnp.float32),
        ],
        interpret=interpret,
    )(x_flat, wr)


def _dispatch(x_flat, gates, scat_idx3, n_slot):
    n_tok, d = x_flat.shape
    per_w = n_tok // _SC_WORKERS
    chunk = min(_SC_CHUNK, per_w)
    n_chunk = per_w // chunk
    mesh = plsc.VectorSubcoreMesh(core_axis_name="c", subcore_axis_name="s")

    @functools.partial(
        pl.kernel,
        out_type=[
            jax.ShapeDtypeStruct((n_slot, d), jnp.float32),
            jax.ShapeDtypeStruct((n_slot, _GATE_W), jnp.float32),
        ],
        mesh=mesh,
        scratch_types=[
            pltpu.VMEM((n_chunk, chunk), jnp.int32),
            pltpu.VMEM((chunk, d), jnp.float32),
            pltpu.VMEM((chunk, d), jnp.float32),
            pltpu.VMEM((chunk, _GATE_W), jnp.float32),
            pltpu.VMEM((chunk, _GATE_W), jnp.float32),
            pltpu.SemaphoreType.DMA,
            pltpu.SemaphoreType.DMA,
            pltpu.SemaphoreType.DMA,
            pltpu.SemaphoreType.DMA,
        ],
    )
    def k(x_hbm, g_hbm, idx_hbm, xg_hbm, gs_hbm, idx_v,
          buf_a, buf_b, gbuf_a, gbuf_b, sia, sib, soa, sob):
        wid = lax.axis_index("s") * 2 + lax.axis_index("c")
        base = wid * per_w
        pltpu.sync_copy(idx_hbm.at[wid], idx_v)

        bufs = (buf_a, buf_b)
        gbufs = (gbuf_a, gbuf_b)
        sin = (sia, sib)
        sout = (soa, sob)

        def cp_in(c):
            sl = pl.ds(base + c * chunk, chunk)
            h1 = pltpu.async_copy(x_hbm.at[sl], bufs[c % 2], sin[c % 2])
            h2 = pltpu.async_copy(g_hbm.at[sl], gbufs[c % 2], sin[c % 2])
            return (h1, h2)

        def cp_out(c):
            h1 = pltpu.async_copy(bufs[c % 2], xg_hbm.at[idx_v.at[c]],
                                  sout[c % 2])
            h2 = pltpu.async_copy(gbufs[c % 2], gs_hbm.at[idx_v.at[c]],
                                  sout[c % 2])
            return (h1, h2)

        def wait(hs):
            for h in hs:
                h.wait()

        pend_in = {0: cp_in(0)}
        pend_out = {}
        for c in range(n_chunk):
            wait(pend_in.pop(c))
            if c + 1 < n_chunk:
                if c >= 1:
                    wait(pend_out.pop(c - 1))
                pend_in[c + 1] = cp_in(c + 1)
            pend_out[c] = cp_out(c)
        for c in sorted(pend_out):
            wait(pend_out.pop(c))

    return k(x_flat, gates, scat_idx3)


def _collect(yg, gath_idx3, n_tok):
    d = yg.shape[1]
    per_w = n_tok // _SC_WORKERS
    chunk = min(_SC_CHUNK, per_w)
    n_chunk = per_w // chunk
    mesh = plsc.VectorSubcoreMesh(core_axis_name="c", subcore_axis_name="s")

    @functools.partial(
        pl.kernel,
        out_type=jax.ShapeDtypeStruct((n_tok, d), jnp.float32),
        mesh=mesh,
        scratch_types=[
            pltpu.VMEM((n_chunk, chunk), jnp.int32),
            pltpu.VMEM((chunk, d), jnp.float32),
            pltpu.VMEM((chunk, d), jnp.float32),
            pltpu.SemaphoreType.DMA,
            pltpu.SemaphoreType.DMA,
            pltpu.SemaphoreType.DMA,
            pltpu.SemaphoreType.DMA,
        ],
    )
    def k(yg_hbm, idx_hbm, ys_hbm, idx_v, buf_a, buf_b, sia, sib, soa, sob):
        wid = lax.axis_index("s") * 2 + lax.axis_index("c")
        base = wid * per_w
        pltpu.sync_copy(idx_hbm.at[wid], idx_v)

        bufs = (buf_a, buf_b)
        sin = (sia, sib)
        sout = (soa, sob)

        def cp_in(c):
            return pltpu.async_copy(
                yg_hbm.at[idx_v.at[c]], bufs[c % 2], sin[c % 2])

        def cp_out(c):
            return pltpu.async_copy(
                bufs[c % 2], ys_hbm.at[pl.ds(base + c * chunk, chunk)],
                sout[c % 2])

        pend_in = {0: cp_in(0)}
        pend_out = {}
        for c in range(n_chunk):
            pend_in.pop(c).wait()
            if c + 1 < n_chunk:
                if c >= 1:
                    pend_out.pop(c - 1).wait()
                pend_in[c + 1] = cp_in(c + 1)
            pend_out[c] = cp_out(c)
        for c in sorted(pend_out):
            pend_out.pop(c).wait()

    return k(yg, gath_idx3)


def _ffn_body(n_exp, d, cap, epb, xg_ref, gs_ref, wg_ref, wu_ref, wd_ref,
              yg_ref):
    e = pl.program_id(0)

    @pl.when(e < n_exp // epb)
    def _():
        for k in range(epb):
            rows = pl.ds(k * cap, cap)
            xe = xg_ref[rows, :].astype(jnp.bfloat16)
            gate = gs_ref[rows, 0:1]
            a = jnp.dot(xe, wg_ref[k].astype(jnp.bfloat16),
                        preferred_element_type=jnp.float32)
            b = jnp.dot(xe, wu_ref[k].astype(jnp.bfloat16),
                        preferred_element_type=jnp.float32)
            h = (a * jax.nn.sigmoid(a)) * b
            y = jnp.dot(h.astype(jnp.bfloat16),
                        wd_ref[k].astype(jnp.bfloat16),
                        preferred_element_type=jnp.float32)
            yg_ref[rows, :] = y * gate

    @pl.when(e == n_exp // epb)
    def _():
        yg_ref[...] = jnp.zeros_like(yg_ref)


def _ffn(xg, gs, wg, wu, wd, cap, interpret=False):
    n_exp, d, dff = wg.shape
    epb = 2
    body = functools.partial(_ffn_body, n_exp, d, cap, epb)
    last = n_exp // epb - 1
    return pl.pallas_call(
        body,
        grid=(n_exp // epb + 1,),
        in_specs=[
            pl.BlockSpec((epb * cap, d), lambda e: (e, 0)),
            pl.BlockSpec((epb * cap, _GATE_W), lambda e: (e, 0)),
            pl.BlockSpec((epb, d, dff), lambda e: (jnp.minimum(e, last), 0, 0)),
            pl.BlockSpec((epb, d, dff), lambda e: (jnp.minimum(e, last), 0, 0)),
            pl.BlockSpec((epb, dff, d), lambda e: (jnp.minimum(e, last), 0, 0)),
        ],
        out_specs=pl.BlockSpec((epb * cap, d), lambda e: (e, 0)),
        out_shape=jax.ShapeDtypeStruct(((n_exp + epb) * cap, d), jnp.float32),
        interpret=interpret,
    )(xg, gs, wg, wu, wd)


def kernel(x, Wr, Wg, Wu, Wd):
    b, s, d = x.shape
    n_tok = b * s
    n_exp = Wr.shape[1]
    cap = max(1, int(math.ceil(1.25 * (n_tok / n_exp))))
    x_flat = x.reshape(n_tok, d)

    gates, scat2, aux = _router(x_flat, Wr, cap, 0.01)
    per_w = n_tok // _SC_WORKERS
    chunk = min(_SC_CHUNK, per_w)
    scat3 = scat2.reshape(_SC_WORKERS, per_w // chunk, chunk)
    gath3 = scat3

    n_slot = (n_exp + 2) * cap
    xg, gs = _dispatch(x_flat, gates, scat3, n_slot)
    yg = _ffn(xg, gs, Wg, Wu, Wd, cap)
    out = _collect(yg, gath3, n_tok)
    return out.reshape(b, s, d), aux.reshape(())
